# Initial kernel scaffold; baseline (speedup 1.0000x reference)
import jax
import jax.numpy as jnp
from jax.experimental import pallas as pl


def kernel(dmap_conv, dmap_tran, gt_density, process):
    raise NotImplementedError("write your pallas kernel here")


# speedup vs baseline: 4.0805x; 4.0805x over previous
import jax
import jax.numpy as jnp
from jax import lax
from jax.experimental import pallas as pl
from jax.experimental.pallas import tpu as pltpu

_B = 32
_H = 64
_W = 64
_GH = 512
_GW = 512
_POOL = 8


def _body(num_ref, wgt_ref, conv_ref, tran_ref, gt_ref, main_ref, fb_ref,
          pooled_ref):
    b = pl.program_id(0)

    g = gt_ref[0]
    rows = lax.broadcasted_iota(jnp.int32, (_GH, _H), 0)
    cols = lax.broadcasted_iota(jnp.int32, (_GH, _H), 1)
    pmat = jnp.where(rows // _POOL == cols, 1.0, 0.0).astype(jnp.float32)
    a = lax.dot_general(g, pmat, (((1,), (0,)), ((), ())),
                        precision=lax.Precision.HIGHEST)
    pooled = lax.dot_general(pmat, a, (((0,), (0,)), ((), ())),
                             precision=lax.Precision.HIGHEST)
    pooled_ref[b] = pooled

    @pl.when(b == _B - 1)
    def _():
        conv = conv_ref[...]
        tran = tran_ref[...]
        gt = pooled_ref[...]
        k = jnp.maximum(num_ref[0], 1)
        w = wgt_ref[0]

        e1 = jnp.abs(gt - conv)
        e2 = jnp.abs(gt - tran)
        bits1 = lax.bitcast_convert_type(e1, jnp.int32)
        bits2 = lax.bitcast_convert_type(e2, jnp.int32)

        def step(i, carry):
            p1, p2 = carry
            bit = jnp.left_shift(jnp.int32(1), 30 - i)
            c1 = p1 | bit
            cnt1 = jnp.sum((bits1 >= c1).astype(jnp.int32), axis=(1, 2),
                           keepdims=True)
            p1 = jnp.where(cnt1 >= k, c1, p1)
            c2 = p2 | bit
            cnt2 = jnp.sum((bits2 >= c2).astype(jnp.int32), axis=(1, 2),
                           keepdims=True)
            p2 = jnp.where(cnt2 >= k, c2, p2)
            return (p1, p2)

        zero = jnp.zeros((_B, 1, 1), jnp.int32)
        t1, t2 = lax.fori_loop(0, 31, step, (zero, zero))

        mask1 = bits1 >= t1
        mask2 = bits2 >= t2
        comb_tran = w * tran + (1.0 - w) * gt
        comb_conv = w * conv + (1.0 - w) * gt
        d_cg = conv - gt
        d_tg = tran - gt
        fb_ref[0] = jnp.sum(d_cg * d_cg) + jnp.sum(d_tg * d_tg)
        m1 = jnp.where(mask1, conv - comb_tran, d_cg)
        m2 = jnp.where(mask2, tran - comb_conv, d_tg)
        main_ref[0] = jnp.sum(m1 * m1) + jnp.sum(m2 * m2)


def kernel(dmap_conv, dmap_tran, gt_density, process):
    conv = dmap_conv.reshape(_B, _H, _W)
    tran = dmap_tran.reshape(_B, _H, _W)
    gt = gt_density.reshape(_B, _GH, _GW)
    p = process.astype(jnp.float32)
    num = jnp.floor((_H * _W) * (0.1 * p)).astype(jnp.int32)
    wgt = 1.0 * p

    main, fb = pl.pallas_call(
        _body,
        grid=(_B,),
        in_specs=[
            pl.BlockSpec(memory_space=pltpu.SMEM),
            pl.BlockSpec(memory_space=pltpu.SMEM),
            pl.BlockSpec((_B, _H, _W), lambda b: (0, 0, 0)),
            pl.BlockSpec((_B, _H, _W), lambda b: (0, 0, 0)),
            pl.BlockSpec((1, _GH, _GW), lambda b: (b, 0, 0)),
        ],
        out_specs=[
            pl.BlockSpec(memory_space=pltpu.SMEM),
            pl.BlockSpec(memory_space=pltpu.SMEM),
        ],
        out_shape=[
            jax.ShapeDtypeStruct((1,), jnp.float32),
            jax.ShapeDtypeStruct((1,), jnp.float32),
        ],
        scratch_shapes=[pltpu.VMEM((_B, _H, _W), jnp.float32)],
        compiler_params=pltpu.CompilerParams(
            dimension_semantics=("arbitrary",)),
    )(num, wgt, conv, tran, gt)

    return jnp.where(num[0] < 1, fb[0], main[0])


# speedup vs baseline: 7.4480x; 1.8253x over previous
import jax
import jax.numpy as jnp
from jax import lax
from jax.experimental import pallas as pl
from jax.experimental.pallas import tpu as pltpu

_B = 32
_N = 4096
_GH = 512
_GW = 512
_POOL = 8
_SLAB = 4
_STEPS = _B // _SLAB


def _body(num_ref, wgt_ref, conv_ref, tran_ref, gt_ref, main_ref, fb_ref,
          pooled_ref):
    b = pl.program_id(0)

    ri = lax.broadcasted_iota(jnp.int32, (_GH, 64), 0)
    ci = lax.broadcasted_iota(jnp.int32, (_GH, 64), 1)
    pmat = jnp.where(ri // _POOL == ci, 1.0, 0.0).astype(jnp.float32)
    ui = lax.broadcasted_iota(jnp.int32, (32, _GH), 0)
    ii = lax.broadcasted_iota(jnp.int32, (32, _GH), 1)
    l0 = jnp.where(ii // _POOL == 2 * ui, 1.0, 0.0).astype(jnp.float32)
    l1 = jnp.where(ii // _POOL == 2 * ui + 1, 1.0, 0.0).astype(jnp.float32)

    for s in range(_SLAB):
        g = gt_ref[s]
        a = lax.dot_general(g, pmat, (((1,), (0,)), ((), ())))
        q0 = lax.dot_general(l0, a, (((1,), (0,)), ((), ())))
        q1 = lax.dot_general(l1, a, (((1,), (0,)), ((), ())))
        pooled_ref[b * _SLAB + s] = jnp.concatenate([q0, q1], axis=1)

    @pl.when(b == _STEPS - 1)
    def _():
        conv = conv_ref[...]
        tran = tran_ref[...]
        gt = pooled_ref[...]
        k = jnp.maximum(num_ref[0], 1)
        w = wgt_ref[0]

        e1 = jnp.abs(gt - conv)
        e2 = jnp.abs(gt - tran)
        bits1 = lax.bitcast_convert_type(e1, jnp.int32)
        bits2 = lax.bitcast_convert_type(e2, jnp.int32)

        def step(i, carry):
            p1, p2 = carry
            bit = jnp.left_shift(jnp.int32(1), 30 - i)
            c1 = p1 | bit
            cnt1 = jnp.sum((bits1 >= c1).astype(jnp.int32), axis=(1, 2),
                           keepdims=True)
            p1 = jnp.where(cnt1 >= k, c1, p1)
            c2 = p2 | bit
            cnt2 = jnp.sum((bits2 >= c2).astype(jnp.int32), axis=(1, 2),
                           keepdims=True)
            p2 = jnp.where(cnt2 >= k, c2, p2)
            return (p1, p2)

        zero = jnp.zeros((_B, 1, 1), jnp.int32)
        t1, t2 = lax.fori_loop(0, 31, step, (zero, zero))

        mask1 = bits1 >= t1
        mask2 = bits2 >= t2
        comb_tran = w * tran + (1.0 - w) * gt
        comb_conv = w * conv + (1.0 - w) * gt
        d_cg = conv - gt
        d_tg = tran - gt
        fb_ref[0] = jnp.sum(d_cg * d_cg) + jnp.sum(d_tg * d_tg)
        m1 = jnp.where(mask1, conv - comb_tran, d_cg)
        m2 = jnp.where(mask2, tran - comb_conv, d_tg)
        main_ref[0] = jnp.sum(m1 * m1) + jnp.sum(m2 * m2)


def kernel(dmap_conv, dmap_tran, gt_density, process):
    conv = dmap_conv.reshape(_B, 32, 128)
    tran = dmap_tran.reshape(_B, 32, 128)
    gt = gt_density.reshape(_B, _GH, _GW)
    p = process.astype(jnp.float32)
    num = jnp.floor(_N * (0.1 * p)).astype(jnp.int32)
    wgt = 1.0 * p

    main, fb = pl.pallas_call(
        _body,
        grid=(_STEPS,),
        in_specs=[
            pl.BlockSpec(memory_space=pltpu.SMEM),
            pl.BlockSpec(memory_space=pltpu.SMEM),
            pl.BlockSpec((_B, 32, 128), lambda b: (0, 0, 0)),
            pl.BlockSpec((_B, 32, 128), lambda b: (0, 0, 0)),
            pl.BlockSpec((_SLAB, _GH, _GW), lambda b: (b, 0, 0)),
        ],
        out_specs=[
            pl.BlockSpec(memory_space=pltpu.SMEM),
            pl.BlockSpec(memory_space=pltpu.SMEM),
        ],
        out_shape=[
            jax.ShapeDtypeStruct((1,), jnp.float32),
            jax.ShapeDtypeStruct((1,), jnp.float32),
        ],
        scratch_shapes=[pltpu.VMEM((_B, 32, 128), jnp.float32)],
        compiler_params=pltpu.CompilerParams(
            dimension_semantics=("arbitrary",)),
    )(num, wgt, conv, tran, gt)

    return jnp.where(num[0] < 1, fb[0], main[0])


# speedup vs baseline: 7.9445x; 1.0667x over previous
import jax
import jax.numpy as jnp
from jax import lax
from jax.experimental import pallas as pl
from jax.experimental.pallas import tpu as pltpu

_B = 32
_N = 4096
_GH = 512
_GW = 512
_POOL = 8
_SLAB = 4
_STEPS = _B // _SLAB


def _body(num_ref, wgt_ref, conv_ref, tran_ref, gt_ref, main_ref, fb_ref,
          pooled_ref):
    b = pl.program_id(0)

    g_all = gt_ref[...]
    gr = jnp.sum(jnp.reshape(g_all, (_SLAB, 64, _POOL, _GW)), axis=2)
    gr2 = jnp.reshape(gr, (_SLAB * 64, _GW))
    ri = lax.broadcasted_iota(jnp.int32, (_GW, 64), 0)
    ci = lax.broadcasted_iota(jnp.int32, (_GW, 64), 1)
    pmat = jnp.where(ri // _POOL == ci, 1.0, 0.0).astype(jnp.float32)
    rp = lax.dot_general(gr2, pmat, (((1,), (0,)), ((), ())))
    oi = lax.broadcasted_iota(jnp.int32, (128, 256), 0)
    rj = lax.broadcasted_iota(jnp.int32, (128, 256), 1)
    sel = 64 * (oi // 32) + 2 * (oi % 32)
    l0 = jnp.where(rj == sel, 1.0, 0.0).astype(jnp.float32)
    l1 = jnp.where(rj == sel + 1, 1.0, 0.0).astype(jnp.float32)
    q0 = lax.dot_general(l0, rp, (((1,), (0,)), ((), ())))
    q1 = lax.dot_general(l1, rp, (((1,), (0,)), ((), ())))
    qq = jnp.concatenate([q0, q1], axis=1)
    pooled_ref[pl.ds(b * _SLAB, _SLAB)] = jnp.reshape(qq, (_SLAB, 32, 128))

    @pl.when(b == _STEPS - 1)
    def _():
        conv = conv_ref[...]
        tran = tran_ref[...]
        gt = pooled_ref[...]
        k = jnp.maximum(num_ref[0], 1)
        w = wgt_ref[0]

        e1 = jnp.abs(gt - conv)
        e2 = jnp.abs(gt - tran)
        bits1 = lax.bitcast_convert_type(e1, jnp.int32)
        bits2 = lax.bitcast_convert_type(e2, jnp.int32)

        def step(i, carry):
            p1, p2 = carry
            bit = jnp.left_shift(jnp.int32(1), 30 - i)
            c1 = p1 | bit
            cnt1 = jnp.sum((bits1 >= c1).astype(jnp.int32), axis=(1, 2),
                           keepdims=True)
            p1 = jnp.where(cnt1 >= k, c1, p1)
            c2 = p2 | bit
            cnt2 = jnp.sum((bits2 >= c2).astype(jnp.int32), axis=(1, 2),
                           keepdims=True)
            p2 = jnp.where(cnt2 >= k, c2, p2)
            return (p1, p2)

        zero = jnp.zeros((_B, 1, 1), jnp.int32)
        t1, t2 = lax.fori_loop(0, 31, step, (zero, zero))

        mask1 = bits1 >= t1
        mask2 = bits2 >= t2
        comb_tran = w * tran + (1.0 - w) * gt
        comb_conv = w * conv + (1.0 - w) * gt
        d_cg = conv - gt
        d_tg = tran - gt
        fb_ref[0] = jnp.sum(d_cg * d_cg) + jnp.sum(d_tg * d_tg)
        m1 = jnp.where(mask1, conv - comb_tran, d_cg)
        m2 = jnp.where(mask2, tran - comb_conv, d_tg)
        main_ref[0] = jnp.sum(m1 * m1) + jnp.sum(m2 * m2)


def kernel(dmap_conv, dmap_tran, gt_density, process):
    conv = dmap_conv.reshape(_B, 32, 128)
    tran = dmap_tran.reshape(_B, 32, 128)
    gt = gt_density.reshape(_B, _GH, _GW)
    p = process.astype(jnp.float32)
    num = jnp.floor(_N * (0.1 * p)).astype(jnp.int32)
    wgt = 1.0 * p

    main, fb = pl.pallas_call(
        _body,
        grid=(_STEPS,),
        in_specs=[
            pl.BlockSpec(memory_space=pltpu.SMEM),
            pl.BlockSpec(memory_space=pltpu.SMEM),
            pl.BlockSpec((_B, 32, 128), lambda b: (0, 0, 0)),
            pl.BlockSpec((_B, 32, 128), lambda b: (0, 0, 0)),
            pl.BlockSpec((_SLAB, _GH, _GW), lambda b: (b, 0, 0)),
        ],
        out_specs=[
            pl.BlockSpec(memory_space=pltpu.SMEM),
            pl.BlockSpec(memory_space=pltpu.SMEM),
        ],
        out_shape=[
            jax.ShapeDtypeStruct((1,), jnp.float32),
            jax.ShapeDtypeStruct((1,), jnp.float32),
        ],
        scratch_shapes=[pltpu.VMEM((_B, 32, 128), jnp.float32)],
        compiler_params=pltpu.CompilerParams(
            dimension_semantics=("arbitrary",)),
    )(num, wgt, conv, tran, gt)

    return jnp.where(num[0] < 1, fb[0], main[0])


# speedup vs baseline: 8.5301x; 1.0737x over previous
import jax
import jax.numpy as jnp
from jax import lax
from jax.experimental import pallas as pl
from jax.experimental.pallas import tpu as pltpu

_B = 32
_N = 4096
_GH = 512
_GW = 512
_POOL = 8
_SLAB = 4
_STEPS = _B // _SLAB


def _body(num_ref, wgt_ref, conv_ref, tran_ref, gt_ref, main_ref, fb_ref,
          pooled_ref):
    b = pl.program_id(0)

    g2 = jnp.reshape(gt_ref[...], (_SLAB * _GH, _GW))
    ri = lax.broadcasted_iota(jnp.int32, (_GW, 64), 0)
    ci = lax.broadcasted_iota(jnp.int32, (_GW, 64), 1)
    pmat = jnp.where(ri // _POOL == ci, 1.0, 0.0).astype(jnp.float32)
    a = lax.dot_general(g2, pmat, (((1,), (0,)), ((), ())))
    s = jnp.sum(jnp.reshape(a, (256, _POOL, 64)), axis=1)
    s2 = jnp.reshape(s, (128, 2, 64))
    qq = jnp.concatenate([s2[:, 0, :], s2[:, 1, :]], axis=1)
    pooled_ref[pl.ds(b * _SLAB, _SLAB)] = jnp.reshape(qq, (_SLAB, 32, 128))

    @pl.when(b == _STEPS - 1)
    def _():
        conv = conv_ref[...]
        tran = tran_ref[...]
        gt = pooled_ref[...]
        k = jnp.maximum(num_ref[0], 1)
        w = wgt_ref[0]

        e1 = jnp.abs(gt - conv)
        e2 = jnp.abs(gt - tran)
        bits1 = lax.bitcast_convert_type(e1, jnp.int32)
        bits2 = lax.bitcast_convert_type(e2, jnp.int32)

        def step(i, carry):
            p1, p2 = carry
            bit = jnp.left_shift(jnp.int32(1), 30 - i)
            c1 = p1 | bit
            cnt1 = jnp.sum((bits1 >= c1).astype(jnp.int32), axis=(1, 2),
                           keepdims=True)
            p1 = jnp.where(cnt1 >= k, c1, p1)
            c2 = p2 | bit
            cnt2 = jnp.sum((bits2 >= c2).astype(jnp.int32), axis=(1, 2),
                           keepdims=True)
            p2 = jnp.where(cnt2 >= k, c2, p2)
            return (p1, p2)

        zero = jnp.zeros((_B, 1, 1), jnp.int32)
        t1, t2 = lax.fori_loop(0, 31, step, (zero, zero))

        mask1 = bits1 >= t1
        mask2 = bits2 >= t2
        comb_tran = w * tran + (1.0 - w) * gt
        comb_conv = w * conv + (1.0 - w) * gt
        d_cg = conv - gt
        d_tg = tran - gt
        fb_ref[0] = jnp.sum(d_cg * d_cg) + jnp.sum(d_tg * d_tg)
        m1 = jnp.where(mask1, conv - comb_tran, d_cg)
        m2 = jnp.where(mask2, tran - comb_conv, d_tg)
        main_ref[0] = jnp.sum(m1 * m1) + jnp.sum(m2 * m2)


def kernel(dmap_conv, dmap_tran, gt_density, process):
    conv = dmap_conv.reshape(_B, 32, 128)
    tran = dmap_tran.reshape(_B, 32, 128)
    gt = gt_density.reshape(_B, _GH, _GW)
    p = process.astype(jnp.float32)
    num = jnp.floor(_N * (0.1 * p)).astype(jnp.int32)
    wgt = 1.0 * p

    main, fb = pl.pallas_call(
        _body,
        grid=(_STEPS,),
        in_specs=[
            pl.BlockSpec(memory_space=pltpu.SMEM),
            pl.BlockSpec(memory_space=pltpu.SMEM),
            pl.BlockSpec((_B, 32, 128), lambda b: (0, 0, 0)),
            pl.BlockSpec((_B, 32, 128), lambda b: (0, 0, 0)),
            pl.BlockSpec((_SLAB, _GH, _GW), lambda b: (b, 0, 0)),
        ],
        out_specs=[
            pl.BlockSpec(memory_space=pltpu.SMEM),
            pl.BlockSpec(memory_space=pltpu.SMEM),
        ],
        out_shape=[
            jax.ShapeDtypeStruct((1,), jnp.float32),
            jax.ShapeDtypeStruct((1,), jnp.float32),
        ],
        scratch_shapes=[pltpu.VMEM((_B, 32, 128), jnp.float32)],
        compiler_params=pltpu.CompilerParams(
            dimension_semantics=("arbitrary",)),
    )(num, wgt, conv, tran, gt)

    return jnp.where(num[0] < 1, fb[0], main[0])
